# Initial kernel scaffold; baseline (speedup 1.0000x reference)
import jax
import jax.numpy as jnp
from jax.experimental import pallas as pl


def kernel(points_2d, image_indices, camera_indices, point_indices, camera_pps, depths_ref, extrinsics, intrinsics, points_3d):
    raise NotImplementedError("write your pallas kernel here")


# speedup vs baseline: 16.5238x; 16.5238x over previous
import functools

import jax
import jax.numpy as jnp
from jax import lax
from jax.experimental import pallas as pl
from jax.experimental.pallas import tpu as pltpu
from jax.experimental.pallas import tpu_sc as plsc

NC = 2
NS = 16
NW = NC * NS
LANES = 16

CHUNK = 3968
KCHUNK = 8
NPAD = NW * KCHUNK * CHUNK
EXT_WORDS = 2000 * 8


def _sc_body(p2dx, p2dy, ptidx, imidx, dep, par, px, py, pz, ext,
             uo_out, vo_out, do_out,
             ptidx_v, imidx_v, pxv, pyv, pzv, oxv, oyv, depv,
             uov, vov, dov, ext_v, par_v, sem1, sem2, sem3):
    wid = lax.axis_index("s") * NC + lax.axis_index("c")
    pltpu.sync_copy(par, par_v)
    pltpu.sync_copy(ext, ext_v)
    fx = par_v[0]
    fy = par_v[1]
    k1 = par_v[2]
    k2 = par_v[3]
    ppx = par_v[4]
    ppy = par_v[5]

    def chunk_body(k, carry):
        base = wid * (KCHUNK * CHUNK) + k * CHUNK
        pltpu.sync_copy(ptidx.at[pl.ds(base, CHUNK)], ptidx_v)
        pltpu.sync_copy(imidx.at[pl.ds(base, CHUNK)], imidx_v)
        cp1 = pltpu.async_copy(px.at[ptidx_v], pxv, sem1)
        cp2 = pltpu.async_copy(py.at[ptidx_v], pyv, sem2)
        cp3 = pltpu.async_copy(pz.at[ptidx_v], pzv, sem3)
        pltpu.sync_copy(p2dx.at[pl.ds(base, CHUNK)], oxv)
        pltpu.sync_copy(p2dy.at[pl.ds(base, CHUNK)], oyv)
        pltpu.sync_copy(dep.at[pl.ds(base, CHUNK)], depv)
        cp1.wait()
        cp2.wait()
        cp3.wait()

        def grp(g, gc):
            b = g * LANES
            sl = pl.ds(b, LANES)
            vx = pxv[sl]
            vy = pyv[sl]
            vz = pzv[sl]
            im8 = imidx_v[sl] * 8
            qw = plsc.load_gather(ext_v, [im8])
            qx = plsc.load_gather(ext_v, [im8 + 1])
            qy = plsc.load_gather(ext_v, [im8 + 2])
            qz = plsc.load_gather(ext_v, [im8 + 3])
            tx = plsc.load_gather(ext_v, [im8 + 4])
            ty = plsc.load_gather(ext_v, [im8 + 5])
            tz = plsc.load_gather(ext_v, [im8 + 6])
            ox = oxv[sl]
            oy = oyv[sl]
            dref = depv[sl]

            s = qw * qw + qx * qx + qy * qy + qz * qz
            uvx = qy * vz - qz * vy
            uvy = qz * vx - qx * vz
            uvz = qx * vy - qy * vx
            uux = qy * uvz - qz * uvy
            uuy = qz * uvx - qx * uvz
            uuz = qx * uvy - qy * uvx
            inv2 = 2.0 / s
            pcx = vx + inv2 * (qw * uvx + uux) + tx
            pcy = vy + inv2 * (qw * uvy + uuy) + ty
            pcz = vz + inv2 * (qw * uvz + uuz) + tz
            rcp = 1.0 / (pcz + 1e-6)
            xn = pcx * rcp
            yn = pcy * rcp
            r2 = xn * xn + yn * yn
            dist = 1.0 + r2 * (k1 + k2 * r2)
            uov[sl] = fx * xn * dist + ppx - ox
            vov[sl] = fy * yn * dist + ppy - oy
            dov[sl] = rcp - dref
            return gc

        lax.fori_loop(0, CHUNK // LANES, grp, 0)
        pltpu.sync_copy(uov, uo_out.at[pl.ds(base, CHUNK)])
        pltpu.sync_copy(vov, vo_out.at[pl.ds(base, CHUNK)])
        pltpu.sync_copy(dov, do_out.at[pl.ds(base, CHUNK)])
        return carry

    lax.fori_loop(0, KCHUNK, chunk_body, 0)


_sc_call = functools.partial(
    pl.kernel,
    out_type=(
        jax.ShapeDtypeStruct((NPAD,), jnp.float32),
        jax.ShapeDtypeStruct((NPAD,), jnp.float32),
        jax.ShapeDtypeStruct((NPAD,), jnp.float32),
    ),
    mesh=plsc.VectorSubcoreMesh(core_axis_name="c", subcore_axis_name="s"),
    compiler_params=pltpu.CompilerParams(needs_layout_passes=False),
    scratch_types=[
        pltpu.VMEM((CHUNK,), jnp.int32),
        pltpu.VMEM((CHUNK,), jnp.int32),
        pltpu.VMEM((CHUNK,), jnp.float32),
        pltpu.VMEM((CHUNK,), jnp.float32),
        pltpu.VMEM((CHUNK,), jnp.float32),
        pltpu.VMEM((CHUNK,), jnp.float32),
        pltpu.VMEM((CHUNK,), jnp.float32),
        pltpu.VMEM((CHUNK,), jnp.float32),
        pltpu.VMEM((CHUNK,), jnp.float32),
        pltpu.VMEM((CHUNK,), jnp.float32),
        pltpu.VMEM((CHUNK,), jnp.float32),
        pltpu.VMEM((EXT_WORDS,), jnp.float32),
        pltpu.VMEM((6, LANES), jnp.float32),
        pltpu.SemaphoreType.DMA,
        pltpu.SemaphoreType.DMA,
        pltpu.SemaphoreType.DMA,
    ],
)(_sc_body)


def kernel(points_2d, image_indices, camera_indices, point_indices,
           camera_pps, depths_ref, extrinsics, intrinsics, points_3d):
    n = points_2d.shape[0]
    padn = NPAD - n
    ptidx = jnp.pad(point_indices.astype(jnp.int32), (0, padn))
    imidx = jnp.pad(image_indices.astype(jnp.int32), (0, padn))
    p2dx = jnp.pad(points_2d[:, 0], (0, padn))
    p2dy = jnp.pad(points_2d[:, 1], (0, padn))
    dep = jnp.pad(depths_ref, (0, padn))
    px = points_3d[:, 0]
    py = points_3d[:, 1]
    pz = points_3d[:, 2]
    ext = jnp.pad(extrinsics, ((0, 0), (0, 1))).reshape(-1)
    par = jnp.tile(
        jnp.concatenate([intrinsics[0], camera_pps[0]])[:, None], (1, LANES))
    uo, vo, do = _sc_call(p2dx, p2dy, ptidx, imidx, dep, par, px, py, pz, ext)
    return jnp.stack([uo[:n], vo[:n], do[:n]], axis=-1)
